# Initial kernel scaffold; baseline (speedup 1.0000x reference)
import jax
import jax.numpy as jnp
from jax.experimental import pallas as pl


def kernel(x, edge_index, batch, W1, b1, W2, b2, lin_W, lin_b):
    raise NotImplementedError("write your pallas kernel here")


# speedup vs baseline: 19.9962x; 19.9962x over previous
import functools

import jax
import jax.numpy as jnp
from jax import lax
from jax.experimental import pallas as pl
from jax.experimental.pallas import tpu as pltpu
from jax.experimental.pallas import tpu_sc as plsc

N = 100000
E = 1600000
G = 256
H = 32

NC = 2
NS = 16
L = 16

HALF = 50048
NPAD = 2 * HALF
TPT = HALF // NS
TRASH = HALF
ACCROWS = HALF + 128
ZSPAN = ACCROWS // NS

KE = 512
NIR = KE // 128
EPT = 100352
NCHUNK = EPT // KE
ZB = 392
EPAD = EPT * NS
ERT = EPT // 128
ER = EPAD // 128
SENT = NPAD

GACC = G + 16

_mesh = plsc.VectorSubcoreMesh(core_axis_name="c", subcore_axis_name="s")
_sc_params = pltpu.CompilerParams(use_tc_tiling_on_sc=False)


def _route(idx_ref, j, base):
    for k in range(128 // L):
        d = idx_ref[j, pl.ds(k * L, L)]
        lo = d - base
        ok = lo.astype(jnp.uint32) < jnp.uint32(HALF)
        tr = TRASH + k * L + lax.iota(jnp.int32, L)
        idx_ref[j, pl.ds(k * L, L)] = jnp.where(ok, lo, tr)


def _zero_vmem_2d(ref, rows):
    def zb(r, carry):
        for kk in range(H // L):
            ref[r, pl.ds(kk * L, L)] = jnp.zeros((L,), jnp.float32)
        return carry
    lax.fori_loop(0, rows, zb, 0)


def _deg_body(dst_hbm, deg_hbm, idx_v, ones_v, zb_v, deg_sh):
    c = lax.axis_index("c")
    s = lax.axis_index("s")
    base = c * HALF

    def zb(i, carry):
        zb_v[pl.ds(i * L, L)] = jnp.zeros((L,), jnp.float32)
        return carry
    lax.fori_loop(0, ZSPAN // L, zb, 0)
    pltpu.sync_copy(zb_v, deg_sh.at[pl.ds(s * ZSPAN, ZSPAN)])
    for i in range(128 // L):
        ones_v[pl.ds(i * L, L)] = jnp.ones((L,), jnp.float32)
    plsc.subcore_barrier()

    def body(g, carry):
        row0 = s * ERT + g * NIR
        pltpu.sync_copy(dst_hbm.at[pl.ds(row0, NIR)], idx_v)
        for j in range(NIR):
            _route(idx_v, j, base)
        for j in range(NIR):
            pltpu.sync_copy(ones_v, deg_sh.at[idx_v.at[j]], add=True)
        return carry
    lax.fori_loop(0, NCHUNK, body, 0)

    plsc.subcore_barrier()
    off = s * TPT
    pltpu.sync_copy(deg_sh.at[pl.ds(off, TPT)], zb_v.at[pl.ds(0, TPT)])
    pltpu.sync_copy(zb_v.at[pl.ds(0, TPT)],
                    deg_hbm.at[pl.ds(base + off, TPT)])


_deg_call = functools.partial(
    pl.kernel,
    out_type=jax.ShapeDtypeStruct((NPAD,), jnp.float32),
    mesh=_mesh,
    compiler_params=_sc_params,
    scratch_types=[
        pltpu.VMEM((NIR, 128), jnp.int32),
        pltpu.VMEM((128,), jnp.float32),
        pltpu.VMEM((ZSPAN,), jnp.float32),
        pltpu.VMEM_SHARED((ACCROWS,), jnp.float32),
    ],
)(_deg_body)


def _scatter_body(src_hbm, dst_hbm, xs_hbm, out_hbm,
                  sidx_v, didx_v, rows_v, zrow_v, gsem, acc_sh):
    c = lax.axis_index("c")
    s = lax.axis_index("s")
    base = c * HALF

    _zero_vmem_2d(zrow_v, ZB)
    for t in range(ZSPAN // ZB):
        pltpu.sync_copy(zrow_v, acc_sh.at[pl.ds(s * ZSPAN + t * ZB, ZB)])
    plsc.subcore_barrier()

    def body(g, carry):
        row0 = s * ERT + g * NIR
        pltpu.sync_copy(src_hbm.at[pl.ds(row0, NIR)], sidx_v)
        pltpu.sync_copy(dst_hbm.at[pl.ds(row0, NIR)], didx_v)
        for j in range(NIR):
            _route(didx_v, j, base)
        cps = [pltpu.async_copy(xs_hbm.at[sidx_v.at[j]],
                                rows_v.at[pl.ds(j * 128, 128)], gsem)
               for j in range(NIR)]
        for cp in cps:
            cp.wait()
        for j in range(NIR):
            pltpu.sync_copy(rows_v.at[pl.ds(j * 128, 128)],
                            acc_sh.at[didx_v.at[j]], add=True)
        return carry
    lax.fori_loop(0, NCHUNK, body, 0)

    plsc.subcore_barrier()
    for off0, sz in [(t * ZB, ZB) for t in range(7)] + [(7 * ZB, TPT - 7 * ZB)]:
        off = s * TPT + off0
        pltpu.sync_copy(acc_sh.at[pl.ds(off, sz)], zrow_v.at[pl.ds(0, sz)])
        pltpu.sync_copy(zrow_v.at[pl.ds(0, sz)],
                        out_hbm.at[pl.ds(base + off, sz)])


_scatter_call = functools.partial(
    pl.kernel,
    out_type=jax.ShapeDtypeStruct((NPAD, H), jnp.float32),
    mesh=_mesh,
    compiler_params=_sc_params,
    scratch_types=[
        pltpu.VMEM((NIR, 128), jnp.int32),
        pltpu.VMEM((NIR, 128), jnp.int32),
        pltpu.VMEM((KE, H), jnp.float32),
        pltpu.VMEM((ZB, H), jnp.float32),
        pltpu.SemaphoreType.DMA,
        pltpu.VMEM_SHARED((ACCROWS, H), jnp.float32),
    ],
)(_scatter_body)


def _pool_body(batch_hbm, h_hbm, s_out, c_out,
               bidx_v, rows_v, ones_v, zacc_v, zcnt_v, acc_sh, cnt_sh):
    c = lax.axis_index("c")
    s = lax.axis_index("s")
    node0 = c * HALF + s * TPT

    for i in range(128 // L):
        ones_v[pl.ds(i * L, L)] = jnp.ones((L,), jnp.float32)

    @pl.when(s == 0)
    def _():
        _zero_vmem_2d(zacc_v, GACC)
        def zb(i, carry):
            zcnt_v[pl.ds(i * L, L)] = jnp.zeros((L,), jnp.float32)
            return carry
        lax.fori_loop(0, GACC // L, zb, 0)
        pltpu.sync_copy(zacc_v, acc_sh)
        pltpu.sync_copy(zcnt_v, cnt_sh)
    plsc.subcore_barrier()

    nfull = TPT // 128
    tail = TPT - nfull * 128

    def body(g, carry):
        nb = node0 + g * 128
        pltpu.sync_copy(batch_hbm.at[pl.ds(nb, 128)], bidx_v)
        pltpu.sync_copy(h_hbm.at[pl.ds(nb, 128)], rows_v)
        pltpu.sync_copy(rows_v, acc_sh.at[bidx_v], add=True)
        pltpu.sync_copy(ones_v, cnt_sh.at[bidx_v], add=True)
        return carry
    lax.fori_loop(0, nfull, body, 0)

    for i in range(128 // L):
        bidx_v[pl.ds(i * L, L)] = G + lax.iota(jnp.int32, L)
    nb = node0 + nfull * 128
    pltpu.sync_copy(batch_hbm.at[pl.ds(nb, tail)], bidx_v.at[pl.ds(0, tail)])
    pltpu.sync_copy(h_hbm.at[pl.ds(nb, tail)], rows_v.at[pl.ds(0, tail)])
    pltpu.sync_copy(rows_v, acc_sh.at[bidx_v], add=True)
    pltpu.sync_copy(ones_v, cnt_sh.at[bidx_v], add=True)

    plsc.subcore_barrier()

    @pl.when(s == 0)
    def _():
        pltpu.sync_copy(acc_sh, zacc_v)
        pltpu.sync_copy(zacc_v, s_out.at[c])
        pltpu.sync_copy(cnt_sh, zcnt_v)
        pltpu.sync_copy(zcnt_v, c_out.at[c])


_pool_call = functools.partial(
    pl.kernel,
    out_type=(jax.ShapeDtypeStruct((NC, GACC, H), jnp.float32),
              jax.ShapeDtypeStruct((NC, GACC), jnp.float32)),
    mesh=_mesh,
    compiler_params=_sc_params,
    scratch_types=[
        pltpu.VMEM((128,), jnp.int32),
        pltpu.VMEM((128, H), jnp.float32),
        pltpu.VMEM((128,), jnp.float32),
        pltpu.VMEM((GACC, H), jnp.float32),
        pltpu.VMEM((GACC,), jnp.float32),
        pltpu.VMEM_SHARED((GACC, H), jnp.float32),
        pltpu.VMEM_SHARED((GACC,), jnp.float32),
    ],
)(_pool_body)


BR = 3128


def _tc1_body(x_ref, w_ref, deg_ref, o_ref):
    dis = lax.rsqrt(deg_ref[...] + 1.0)
    o_ref[...] = jnp.dot(x_ref[...], w_ref[...],
                         preferred_element_type=jnp.float32) * dis


def _tc2_body(acc_ref, xs_ref, deg_ref, w_ref, b_ref, o_ref):
    dis = lax.rsqrt(deg_ref[...] + 1.0)
    h = jnp.maximum((acc_ref[...] + xs_ref[...]) * dis + b_ref[...], 0.0)
    o_ref[...] = jnp.dot(h, w_ref[...],
                         preferred_element_type=jnp.float32) * dis


def _tc3_body(acc_ref, xs_ref, deg_ref, b_ref, o_ref):
    dis = lax.rsqrt(deg_ref[...] + 1.0)
    o_ref[...] = jnp.maximum((acc_ref[...] + xs_ref[...]) * dis + b_ref[...],
                             0.0)


def _tc4_body(s_ref, c_ref, w_ref, b_ref, o_ref):
    ssum = s_ref[0] + s_ref[1]
    cnt = c_ref[0] + c_ref[1]
    gmean = ssum[:G] / jnp.maximum(cnt[:G], 1.0)[:, None]
    o_ref[...] = jnp.dot(gmean, w_ref[...],
                         preferred_element_type=jnp.float32) + b_ref[...]


def _rows_spec(cols):
    return pl.BlockSpec((BR, cols), lambda i: (i, 0))


def _full_spec(shape):
    return pl.BlockSpec(shape, lambda i: tuple(0 for _ in shape))


def _tc1(xp, W1, deg2):
    return pl.pallas_call(
        _tc1_body,
        grid=(NPAD // BR,),
        in_specs=[_rows_spec(3), _full_spec((3, H)), _rows_spec(1)],
        out_specs=_rows_spec(H),
        out_shape=jax.ShapeDtypeStruct((NPAD, H), jnp.float32),
    )(xp, W1, deg2)


def _tc2(acc1, xs1, deg2, W2, b1):
    return pl.pallas_call(
        _tc2_body,
        grid=(NPAD // BR,),
        in_specs=[_rows_spec(H), _rows_spec(H), _rows_spec(1),
                  _full_spec((H, H)), _full_spec((1, H))],
        out_specs=_rows_spec(H),
        out_shape=jax.ShapeDtypeStruct((NPAD, H), jnp.float32),
    )(acc1, xs1, deg2, W2, b1)


def _tc3(acc2, xs2, deg2, b2):
    return pl.pallas_call(
        _tc3_body,
        grid=(NPAD // BR,),
        in_specs=[_rows_spec(H), _rows_spec(H), _rows_spec(1),
                  _full_spec((1, H))],
        out_specs=_rows_spec(H),
        out_shape=jax.ShapeDtypeStruct((NPAD, H), jnp.float32),
    )(acc2, xs2, deg2, b2)


def _tc4(sp, cp, lin_W, lin_b):
    return pl.pallas_call(
        _tc4_body,
        in_specs=[pl.BlockSpec((NC, GACC, H), lambda: (0, 0, 0)),
                  pl.BlockSpec((NC, GACC), lambda: (0, 0)),
                  pl.BlockSpec((H, 2), lambda: (0, 0)),
                  pl.BlockSpec((1, 2), lambda: (0, 0))],
        out_specs=pl.BlockSpec((G, 2), lambda: (0, 0)),
        out_shape=jax.ShapeDtypeStruct((G, 2), jnp.float32),
    )(sp, cp, lin_W, lin_b)


def kernel(x, edge_index, batch, W1, b1, W2, b2, lin_W, lin_b):
    src = edge_index[0].astype(jnp.int32)
    dst = edge_index[1].astype(jnp.int32)
    src2d = jnp.pad(src, (0, EPAD - E)).reshape(ER, 128)
    dst2d = jnp.pad(dst, (0, EPAD - E),
                    constant_values=SENT).reshape(ER, 128)
    batchp = jnp.pad(batch.astype(jnp.int32), (0, NPAD - N),
                     constant_values=G)
    xp = jnp.pad(x, ((0, NPAD - N), (0, 0)))

    deg = _deg_call(dst2d)
    deg2 = deg.reshape(NPAD, 1)
    xs1 = _tc1(xp, W1, deg2)
    acc1 = _scatter_call(src2d, dst2d, xs1)
    xs2 = _tc2(acc1, xs1, deg2, W2, b1.reshape(1, H))
    acc2 = _scatter_call(src2d, dst2d, xs2)
    h2 = _tc3(acc2, xs2, deg2, b2.reshape(1, H))
    sp, cp = _pool_call(batchp, h2)
    return _tc4(sp, cp, lin_W, lin_b.reshape(1, 2))
